# speedup vs baseline: 2.9660x; 2.9660x over previous
import functools

import jax
import jax.numpy as jnp
from jax import lax
from jax.experimental import pallas as pl
from jax.experimental.pallas import tpu as pltpu, tpu_sc as plsc

_LANES = 16


@functools.partial(jax.jit, static_argnums=(2, 3, 4))
def _deinterleave(flat, rev, B, L, D):
    info = plsc.get_sparse_core_info()
    nw = info.num_cores * info.num_subcores
    b_per_w = B // nw
    nbuf = min(6, b_per_w)
    lead = min(4, nbuf - 1) if nbuf > 1 else 0
    mesh = plsc.VectorSubcoreMesh(core_axis_name="c", subcore_axis_name="s")

    @functools.partial(
        pl.kernel,
        out_type=jax.ShapeDtypeStruct((B, L, D), flat.dtype),
        mesh=mesh,
        scratch_types=[
            pltpu.VMEM((L,), jnp.int32),
            pltpu.VMEM((nbuf, L), jnp.int32),
            pltpu.VMEM((nbuf, L, D), flat.dtype),
        ]
        + [pltpu.SemaphoreType.DMA] * (2 * nbuf),
    )
    def k(flat_hbm, rev_hbm, out_hbm, rev_v, idx_v, buf_v, *sems):
        gsem, ssem = sems[:nbuf], sems[nbuf:]
        wid = lax.axis_index("s") * info.num_cores + lax.axis_index("c")
        base_b = wid * b_per_w
        pltpu.sync_copy(rev_hbm, rev_v)

        def start_gather(j):
            slot = j % nbuf
            off = (base_b + j) * L
            for t in range(L // _LANES):
                sl = pl.ds(t * _LANES, _LANES)
                idx_v[slot, sl] = rev_v[sl] + off
            return pltpu.async_copy(
                flat_hbm.at[idx_v.at[slot]], buf_v.at[slot], gsem[slot])

        gathers = [None] * b_per_w
        stores = [None] * b_per_w
        for j in range(min(lead, b_per_w)):
            gathers[j] = start_gather(j)
        for i in range(b_per_w):
            slot = i % nbuf
            if gathers[i] is None:
                gathers[i] = start_gather(i)
            gathers[i].wait()
            stores[i] = pltpu.async_copy(
                buf_v.at[slot], out_hbm.at[base_b + i], ssem[slot])
            j = i + lead
            if j < b_per_w and gathers[j] is None:
                prev = j - nbuf
                if prev >= 0:
                    stores[prev].wait()
                gathers[j] = start_gather(j)
        for i in range(max(0, b_per_w - nbuf), b_per_w):
            stores[i].wait()

    return k(flat, rev)


def kernel(inputs, reverse_p_array):
    B, L, D = inputs.shape
    flat = inputs.reshape(B * L, D)
    rev = reverse_p_array.astype(jnp.int32)
    return _deinterleave(flat, rev, B, L, D)


# speedup vs baseline: 2.9759x; 1.0033x over previous
import functools

import jax
import jax.numpy as jnp
from jax import lax
from jax.experimental import pallas as pl
from jax.experimental.pallas import tpu as pltpu, tpu_sc as plsc

_LANES = 16


@functools.partial(jax.jit, static_argnums=(2, 3, 4))
def _deinterleave(flat, rev, B, L, D):
    info = plsc.get_sparse_core_info()
    nw = info.num_cores * info.num_subcores
    b_per_w = B // nw
    nbuf = min(7, b_per_w)
    lead = min(5, nbuf - 1) if nbuf > 1 else 0
    mesh = plsc.VectorSubcoreMesh(core_axis_name="c", subcore_axis_name="s")

    @functools.partial(
        pl.kernel,
        out_type=jax.ShapeDtypeStruct((B, L, D), flat.dtype),
        mesh=mesh,
        scratch_types=[
            pltpu.VMEM((L,), jnp.int32),
            pltpu.VMEM((nbuf, L), jnp.int32),
            pltpu.VMEM((nbuf, L, D), flat.dtype),
        ]
        + [pltpu.SemaphoreType.DMA] * (2 * nbuf),
    )
    def k(flat_hbm, rev_hbm, out_hbm, rev_v, idx_v, buf_v, *sems):
        gsem, ssem = sems[:nbuf], sems[nbuf:]
        wid = lax.axis_index("s") * info.num_cores + lax.axis_index("c")
        base_b = wid * b_per_w
        pltpu.sync_copy(rev_hbm, rev_v)

        def start_gather(j):
            slot = j % nbuf
            off = (base_b + j) * L
            for t in range(L // _LANES):
                sl = pl.ds(t * _LANES, _LANES)
                idx_v[slot, sl] = rev_v[sl] + off
            return pltpu.async_copy(
                flat_hbm.at[idx_v.at[slot]], buf_v.at[slot], gsem[slot])

        gathers = [None] * b_per_w
        stores = [None] * b_per_w
        for j in range(min(lead, b_per_w)):
            gathers[j] = start_gather(j)
        for i in range(b_per_w):
            slot = i % nbuf
            if gathers[i] is None:
                gathers[i] = start_gather(i)
            gathers[i].wait()
            stores[i] = pltpu.async_copy(
                buf_v.at[slot], out_hbm.at[base_b + i], ssem[slot])
            j = i + lead
            if j < b_per_w and gathers[j] is None:
                prev = j - nbuf
                if prev >= 0:
                    stores[prev].wait()
                gathers[j] = start_gather(j)
        for i in range(max(0, b_per_w - nbuf), b_per_w):
            stores[i].wait()

    return k(flat, rev)


def kernel(inputs, reverse_p_array):
    B, L, D = inputs.shape
    flat = inputs.reshape(B * L, D)
    rev = reverse_p_array.astype(jnp.int32)
    return _deinterleave(flat, rev, B, L, D)
